# baseline (speedup 1.0000x reference)
import functools

import jax
import jax.numpy as jnp
from jax import lax
from jax.experimental import pallas as pl
from jax.experimental.pallas import tpu as pltpu
from jax.experimental.pallas import tpu_sc as plsc

_HIDDEN = 1024
_NC = 2
_NS = 16
_NW = _NC * _NS
_B = 4 * 8192
_BPW = _B // _NW

_mesh = plsc.VectorSubcoreMesh(core_axis_name="c", subcore_axis_name="s")


@functools.partial(
    pl.kernel,
    mesh=_mesh,
    out_type=jax.ShapeDtypeStruct((_B, _HIDDEN), jnp.float32),
    scratch_types=[
        pltpu.VMEM((_BPW,), jnp.int32),
        pltpu.SemaphoreType.DMA,
    ],
)
def _gather_kernel(ids_hbm, table_hbm, out_hbm, idx_v, sem):
    wid = lax.axis_index("s") * _NC + lax.axis_index("c")
    base = wid * _BPW
    pltpu.sync_copy(ids_hbm.at[pl.ds(base, _BPW)], idx_v)

    def body(g, carry):
        v = idx_v[pl.ds(g * 16, 16)]
        for k in range(16):
            pltpu.async_copy(
                table_hbm.at[pl.ds(v[k], 1)],
                out_hbm.at[pl.ds(base + g * 16 + k, 1)],
                sem,
            )
        return carry

    lax.fori_loop(0, _BPW // 16, body, 0)

    def drain(i, carry):
        pltpu.make_async_copy(
            table_hbm.at[pl.ds(0, 1)], out_hbm.at[pl.ds(base, 1)], sem
        ).wait()
        return carry

    lax.fori_loop(0, _BPW, drain, 0)


def kernel(input_ids, table):
    ids = input_ids.reshape(-1).astype(jnp.int32)
    out = _gather_kernel(ids, table)
    return out.reshape(input_ids.shape + (_HIDDEN,))


# speedup vs baseline: 35.7523x; 35.7523x over previous
import functools

import jax
import jax.numpy as jnp
from jax import lax
from jax.experimental import pallas as pl
from jax.experimental.pallas import tpu as pltpu
from jax.experimental.pallas import tpu_sc as plsc

_HIDDEN = 1024
_NC = 2
_NS = 16
_NW = _NC * _NS
_B = 4 * 8192
_BPW = _B // _NW
_CHUNK = 32
_NCHUNK = _BPW // _CHUNK

_mesh = plsc.VectorSubcoreMesh(core_axis_name="c", subcore_axis_name="s")


@functools.partial(
    pl.kernel,
    mesh=_mesh,
    out_type=jax.ShapeDtypeStruct((_B, _HIDDEN), jnp.float32),
    scratch_types=[
        pltpu.VMEM((_BPW,), jnp.int32),
        pltpu.VMEM((2, _CHUNK, _HIDDEN), jnp.float32),
        pltpu.SemaphoreType.DMA,
        pltpu.SemaphoreType.DMA,
    ],
)
def _gather_kernel(ids_hbm, table_hbm, out_hbm, idx_v, rows_v, gsem, osem):
    wid = lax.axis_index("s") * _NC + lax.axis_index("c")
    base = wid * _BPW
    pltpu.sync_copy(ids_hbm.at[pl.ds(base, _BPW)], idx_v)

    def gather(j):
        jc = jnp.minimum(j, _NCHUNK - 1)
        pltpu.async_copy(
            table_hbm.at[idx_v.at[pl.ds(jc * _CHUNK, _CHUNK)]],
            rows_v.at[j % 2],
            gsem,
        )

    gather(0)
    gather(1)

    def body(j, carry):
        buf = rows_v.at[j % 2]
        pltpu.make_async_copy(table_hbm.at[pl.ds(0, _CHUNK)], buf, gsem).wait()
        pltpu.async_copy(buf, out_hbm.at[pl.ds(base + j * _CHUNK, _CHUNK)], osem)
        pltpu.make_async_copy(buf, out_hbm.at[pl.ds(base, _CHUNK)], osem).wait()
        gather(j + 2)
        return carry

    lax.fori_loop(0, _NCHUNK, body, 0)

    pltpu.make_async_copy(table_hbm.at[pl.ds(0, _CHUNK)], rows_v.at[0], gsem).wait()
    pltpu.make_async_copy(table_hbm.at[pl.ds(0, _CHUNK)], rows_v.at[1], gsem).wait()


def kernel(input_ids, table):
    ids = input_ids.reshape(-1)
    out = _gather_kernel(ids, table)
    return out.reshape(input_ids.shape + (_HIDDEN,))
